# baseline (speedup 1.0000x reference)
import functools

import jax
import jax.numpy as jnp
from jax import lax
from jax.experimental import pallas as pl
from jax.experimental.pallas import tpu as pltpu
from jax.experimental.pallas import tpu_sc as plsc

NC = 2
NS = 16
L = 16
B = 40


def _sc_body(N, CH, n_echunk, n_fchunk,
             zs, colb_h, rowb_h, outl, outh,
             colv0, rowv0, colv1, rowv1, gbuf, onesb, degb,
             acc, deg, sem0, sem1):
    cid = lax.axis_index("c")
    sid = lax.axis_index("s")
    w = cid * NS + sid
    zero16 = jnp.zeros((L,), jnp.float32)
    ones16 = jnp.ones((L,), jnp.float32)
    g0 = gbuf.at[pl.ds(0, B)]
    g1 = gbuf.at[pl.ds(B, B)]
    slots = ((colv0, rowv0, g0, sem0), (colv1, rowv1, g1, sem1))

    def init_row(r, carry):
        for c8 in range(CH // L):
            gbuf[r, pl.ds(c8 * L, L)] = zero16
        return carry
    lax.fori_loop(0, 2 * B, init_row, 0)

    def init_small(r, carry):
        onesb[pl.ds(r * L, L)] = zero16
        return carry
    lax.fori_loop(0, B // L + 1, init_small, 0)

    def zero_chunk(c, carry):
        ch = sid + NS * c
        @pl.when(ch < N // B)
        def _():
            pltpu.sync_copy(g0, acc.at[pl.ds(ch * B, B)])
            pltpu.sync_copy(onesb.at[pl.ds(0, B)], deg.at[pl.ds(ch * B, B)])
        return carry
    lax.fori_loop(0, (N // B + NS - 1) // NS, zero_chunk, 0)

    def ones_row(r, carry):
        onesb[pl.ds(r * L, L)] = ones16
        return carry
    lax.fori_loop(0, B // L + 1, ones_row, 0)
    plsc.subcore_barrier()

    def load_idx(k, cv, rv):
        pltpu.sync_copy(colb_h.at[w, k], cv)
        pltpu.sync_copy(rowb_h.at[sid, k], rv)

    def start_gather(cv, gb, sem):
        pltpu.async_copy(zs.at[cv.at[0]], gb, sem)

    def drain_slot(cv, rv, gb, sem):
        pltpu.make_async_copy(zs.at[cv.at[0]], gb, sem).wait()
        pltpu.sync_copy(gb, acc.at[rv.at[0]], add=True)
        pltpu.sync_copy(onesb.at[pl.ds(0, B)], deg.at[rv.at[0]], add=True)

    load_idx(0, colv0, rowv0)
    start_gather(colv0, g0, sem0)

    n2 = n_echunk // 2

    def edge_pair(g, carry):
        cv0, rv0, gb0, s0 = slots[0]
        cv1, rv1, gb1, s1 = slots[1]
        load_idx(2 * g + 1, cv1, rv1)
        start_gather(cv1, gb1, s1)
        drain_slot(cv0, rv0, gb0, s0)
        @pl.when(g < n2 - 1)
        def _():
            load_idx(2 * g + 2, cv0, rv0)
            start_gather(cv0, gb0, s0)
        drain_slot(cv1, rv1, gb1, s1)
        return carry
    lax.fori_loop(0, n2, edge_pair, 0)
    plsc.subcore_barrier()

    BF = B
    groups = []
    r0 = 0
    while r0 < BF:
        groups.append((r0, min(L, BF - r0)))
        r0 += L

    def fin_chunk(c, carry):
        ch = sid + NS * c
        @pl.when(ch < n_fchunk)
        def _():
            base = ch * BF
            pltpu.sync_copy(acc.at[pl.ds(base, BF)], g0)
            pltpu.sync_copy(zs.at[pl.ds(cid * N + base, BF)], g1)
            pltpu.sync_copy(deg.at[pl.ds(base, BF)], degb.at[pl.ds(0, BF)])

            for gr0, nrows in groups:
                dv = degb[pl.ds(gr0, L)]
                rdv = 1.0 / jnp.where(dv == 0.0, 1.0, dv)
                for l in range(nrows):
                    r = gr0 + l
                    rd = rdv[l]
                    for c8 in range(CH // L):
                        sl = pl.ds(c8 * L, L)
                        zl = gbuf[r, sl] * rd
                        gbuf[r, sl] = zl
                        gbuf[B + r, sl] = gbuf[B + r, sl] - zl

            pltpu.sync_copy(g0, outl.at[pl.ds(cid * N + base, BF)])
            pltpu.sync_copy(g1, outh.at[pl.ds(cid * N + base, BF)])
        return carry
    lax.fori_loop(0, (n_fchunk + NS - 1) // NS, fin_chunk, 0)


def kernel(Z, edge_index):
    N, C = Z.shape
    E = edge_index.shape[1]
    CH = C // NC
    n_echunk = E // (NS * B)
    n_fchunk = N // B

    row = edge_index[0]
    col = edge_index[1]
    zs = Z.reshape(N, NC, CH).transpose(1, 0, 2).reshape(NC * N, CH)
    col2 = jnp.concatenate([col, col + N]).reshape(NC * NS, n_echunk, 1, B)
    row2 = row.reshape(NS, n_echunk, 1, B)

    body = functools.partial(_sc_body, N, CH, n_echunk, n_fchunk)
    mesh = plsc.VectorSubcoreMesh(core_axis_name="c", subcore_axis_name="s")
    outl, outh = pl.kernel(
        body,
        out_type=(
            jax.ShapeDtypeStruct((NC * N, CH), jnp.float32),
            jax.ShapeDtypeStruct((NC * N, CH), jnp.float32),
        ),
        mesh=mesh,
        scratch_types=(
            pltpu.VMEM((1, B), jnp.int32),
            pltpu.VMEM((1, B), jnp.int32),
            pltpu.VMEM((1, B), jnp.int32),
            pltpu.VMEM((1, B), jnp.int32),
            pltpu.VMEM((2 * B, CH), jnp.float32),
            pltpu.VMEM((B + L,), jnp.float32),
            pltpu.VMEM((B + L,), jnp.float32),
            pltpu.VMEM_SHARED((N, CH), jnp.float32),
            pltpu.VMEM_SHARED((N,), jnp.float32),
            pltpu.SemaphoreType.DMA,
            pltpu.SemaphoreType.DMA,
        ),
        name="spectral_decomposer_sc",
    )(zs, col2, row2)

    z_low = outl.reshape(NC, N, CH).transpose(1, 0, 2).reshape(N, C)
    z_high = outh.reshape(NC, N, CH).transpose(1, 0, 2).reshape(N, C)
    return (z_low, z_high)


# speedup vs baseline: 2.1684x; 2.1684x over previous
import functools

import jax
import jax.numpy as jnp
from jax import lax
from jax.experimental import pallas as pl
from jax.experimental.pallas import tpu as pltpu
from jax.experimental.pallas import tpu_sc as plsc

NC = 2
NS = 16
L = 16
B = 80


def _sc_body(N, CH, n_echunk, n_fchunk,
             zs, colb_h, rowb_h, outl, outh,
             colb, rowb, gbuf, onesb, degb,
             acc, deg, sem0, sem1):
    cid = lax.axis_index("c")
    sid = lax.axis_index("s")
    w = cid * NS + sid
    zero16 = jnp.zeros((L,), jnp.float32)
    ones16 = jnp.ones((L,), jnp.float32)
    g0 = gbuf.at[pl.ds(0, B)]
    g1 = gbuf.at[pl.ds(B, B)]

    def init_row(r, carry):
        for c8 in range(CH // L):
            gbuf[r, pl.ds(c8 * L, L)] = zero16
        return carry
    lax.fori_loop(0, 2 * B, init_row, 0)

    def init_small(r, carry):
        onesb[pl.ds(r * L, L)] = zero16
        return carry
    lax.fori_loop(0, (B + L) // L, init_small, 0)

    ne = colb.shape[0]
    pltpu.sync_copy(colb_h.at[pl.ds(w * ne, ne)], colb)
    pltpu.sync_copy(rowb_h.at[sid], rowb)

    def zero_chunk(c, carry):
        ch = sid + NS * c
        @pl.when(ch < N // B)
        def _():
            pltpu.sync_copy(g0, acc.at[pl.ds(ch * B, B)])
            pltpu.sync_copy(onesb.at[pl.ds(0, B)], deg.at[pl.ds(ch * B, B)])
        return carry
    lax.fori_loop(0, (N // B + NS - 1) // NS, zero_chunk, 0)

    def ones_row(r, carry):
        onesb[pl.ds(r * L, L)] = ones16
        return carry
    lax.fori_loop(0, (B + L) // L, ones_row, 0)
    plsc.subcore_barrier()

    def start_gather(k, gb, sem):
        pltpu.async_copy(zs.at[colb.at[pl.ds(k * B, B)]], gb, sem)

    def drain_slot(k, gb, sem):
        pltpu.make_async_copy(zs.at[colb.at[pl.ds(k * B, B)]], gb, sem).wait()
        pltpu.sync_copy(gb, acc.at[rowb.at[k]], add=True)
        pltpu.sync_copy(onesb.at[pl.ds(0, B)], deg.at[rowb.at[k]], add=True)

    start_gather(0, g0, sem0)
    n2 = n_echunk // 2

    def edge_pair(g, carry):
        start_gather(2 * g + 1, g1, sem1)
        drain_slot(2 * g, g0, sem0)
        start_gather(2 * g + 2, g0, sem0)
        drain_slot(2 * g + 1, g1, sem1)
        return carry
    lax.fori_loop(0, n2, edge_pair, 0)
    drain_slot(n_echunk - 1, g0, sem0)
    plsc.subcore_barrier()

    def fin_chunk(c, carry):
        ch = sid + NS * c
        @pl.when(ch < n_fchunk)
        def _():
            base = ch * B
            pltpu.sync_copy(acc.at[pl.ds(base, B)], g0)
            pltpu.sync_copy(zs.at[pl.ds(cid * N + base, B)], g1)
            pltpu.sync_copy(deg.at[pl.ds(base, B)], degb.at[pl.ds(0, B)])

            for gr0 in range(0, B, L):
                dv = degb[pl.ds(gr0, L)]
                rdv = 1.0 / jnp.where(dv == 0.0, 1.0, dv)
                for l in range(L):
                    r = gr0 + l
                    rd = rdv[l]
                    for c8 in range(CH // L):
                        sl = pl.ds(c8 * L, L)
                        zl = gbuf[r, sl] * rd
                        gbuf[r, sl] = zl
                        gbuf[B + r, sl] = gbuf[B + r, sl] - zl

            pltpu.sync_copy(g0, outl.at[pl.ds(cid * N + base, B)])
            pltpu.sync_copy(g1, outh.at[pl.ds(cid * N + base, B)])
        return carry
    lax.fori_loop(0, (n_fchunk + NS - 1) // NS, fin_chunk, 0)


def kernel(Z, edge_index):
    N, C = Z.shape
    E = edge_index.shape[1]
    CH = C // NC
    n_echunk = E // (NS * B)
    n_fchunk = N // B

    row = edge_index[0]
    col = edge_index[1]
    zs = Z.reshape(N, NC, CH).transpose(1, 0, 2).reshape(NC * N, CH)
    col2 = jnp.concatenate([col, col + N])
    row2 = row.reshape(NS, n_echunk, B)

    body = functools.partial(_sc_body, N, CH, n_echunk, n_fchunk)
    mesh = plsc.VectorSubcoreMesh(core_axis_name="c", subcore_axis_name="s")
    outl, outh = pl.kernel(
        body,
        out_type=(
            jax.ShapeDtypeStruct((NC * N, CH), jnp.float32),
            jax.ShapeDtypeStruct((NC * N, CH), jnp.float32),
        ),
        mesh=mesh,
        scratch_types=(
            pltpu.VMEM((E // NS,), jnp.int32),
            pltpu.VMEM((E // (NS * B), B), jnp.int32),
            pltpu.VMEM((2 * B, CH), jnp.float32),
            pltpu.VMEM((B + L,), jnp.float32),
            pltpu.VMEM((B + L,), jnp.float32),
            pltpu.VMEM_SHARED((N, CH), jnp.float32),
            pltpu.VMEM_SHARED((N,), jnp.float32),
            pltpu.SemaphoreType.DMA,
            pltpu.SemaphoreType.DMA,
        ),
        name="spectral_decomposer_sc",
    )(zs, col2, row2)

    z_low = outl.reshape(NC, N, CH).transpose(1, 0, 2).reshape(N, C)
    z_high = outh.reshape(NC, N, CH).transpose(1, 0, 2).reshape(N, C)
    return (z_low, z_high)
